# baseline (speedup 1.0000x reference)
import functools

import jax
import jax.numpy as jnp
from jax import lax
from jax.experimental import pallas as pl
from jax.experimental.pallas import tpu as pltpu
from jax.experimental.pallas import tpu_sc as plsc

_B, _S, _D = 4, 2048, 768
_NC, _NS = 2, 16
_NW = _NC * _NS
_PPW = _S // _NW
_CH = 32
_NPAIR = _B // 2
_NCHUNK = 4
_LANES = 16

_mesh = plsc.VectorSubcoreMesh(core_axis_name="c", subcore_axis_name="s")


@functools.partial(
    pl.kernel,
    mesh=_mesh,
    out_type=jax.ShapeDtypeStruct((_B, _S, _D), jnp.float32),
    scratch_types=[
        pltpu.VMEM((_B, _PPW), jnp.int32),
        pltpu.VMEM((_CH, _D), jnp.float32),
        pltpu.VMEM((2, 2, _CH, _D), jnp.float32),
        pltpu.SemaphoreType.DMA,
        pltpu.SemaphoreType.DMA,
        pltpu.SemaphoreType.DMA,
        pltpu.SemaphoreType.DMA,
        pltpu.SemaphoreType.DMA,
        pltpu.SemaphoreType.DMA,
    ],
)
def _emb_kernel(tok_hbm, ids_hbm, pos_hbm, out_hbm, idx_v, pos_v, tok_v,
                isem, psem, gsem0, gsem1, ssem0, ssem1):
    wid = lax.axis_index("s") * _NC + lax.axis_index("c")
    p0 = wid * _PPW
    gsem = (gsem0, gsem1)
    ssem = (ssem0, ssem1)

    idx_copies = [
        pltpu.async_copy(ids_hbm.at[b, pl.ds(p0, _PPW)], idx_v.at[b], isem)
        for b in range(_B)
    ]
    pos_copies = [None, None]
    pos_copies[0] = pltpu.async_copy(pos_hbm.at[pl.ds(p0, _CH)], pos_v, psem)
    for h in idx_copies:
        h.wait()

    gh = [None] * _NCHUNK
    sh = [None] * _NCHUNK

    def start_gathers(c):
        k, p = divmod(c, 2)
        r = c % 2
        gh[c] = [
            pltpu.async_copy(
                tok_hbm.at[idx_v.at[2 * p + e, pl.ds(k * _CH, _CH)]],
                tok_v.at[r, e], gsem[r])
            for e in range(2)
        ]

    start_gathers(0)
    pos_copies[0].wait()
    for c in range(_NCHUNK):
        k, p = divmod(c, 2)
        r = c % 2
        for h in gh[c]:
            h.wait()

        @plsc.parallel_loop(0, _CH, unroll=2)
        def _row(j, r=r):
            for cc in range(0, _D, _LANES):
                pv = pos_v[j, pl.ds(cc, _LANES)]
                plsc.addupdate(tok_v.at[r, 0, j, pl.ds(cc, _LANES)], pv)
                plsc.addupdate(tok_v.at[r, 1, j, pl.ds(cc, _LANES)], pv)

        sh[c] = [
            pltpu.async_copy(
                tok_v.at[r, e],
                out_hbm.at[2 * p + e, pl.ds(p0 + k * _CH, _CH)], ssem[r])
            for e in range(2)
        ]
        if c == 1:
            pos_copies[1] = pltpu.async_copy(
                pos_hbm.at[pl.ds(p0 + _CH, _CH)], pos_v, psem)
        if c + 1 < _NCHUNK:
            if c - 1 >= 0:
                for h in sh[c - 1]:
                    h.wait()
            start_gathers(c + 1)
        if c == 1:
            pos_copies[1].wait()
    for c in (_NCHUNK - 2, _NCHUNK - 1):
        for h in sh[c]:
            h.wait()


def kernel(inputs, W_tok, W_pos):
    return _emb_kernel(W_tok, inputs.astype(jnp.int32), W_pos)


# speedup vs baseline: 1.0132x; 1.0132x over previous
import functools

import jax
import jax.numpy as jnp
from jax import lax
from jax.experimental import pallas as pl
from jax.experimental.pallas import tpu as pltpu
from jax.experimental.pallas import tpu_sc as plsc

_B, _S, _D = 4, 2048, 768
_NC, _NS = 2, 16
_NW = _NC * _NS
_PPW = _S // _NW
_CH = 32
_NK = _PPW // _CH
_NCHUNK = _B * _NK
_NBUF = 4
_LANES = 16

_mesh = plsc.VectorSubcoreMesh(core_axis_name="c", subcore_axis_name="s")


@functools.partial(
    pl.kernel,
    mesh=_mesh,
    out_type=jax.ShapeDtypeStruct((_B, _S, _D), jnp.float32),
    scratch_types=[
        pltpu.VMEM((_B, _PPW), jnp.int32),
        pltpu.VMEM((_CH, _D), jnp.float32),
        pltpu.VMEM((_NBUF, _CH, _D), jnp.float32),
    ] + [pltpu.SemaphoreType.DMA] * (2 + 2 * _NBUF),
)
def _emb_kernel(tok_hbm, ids_hbm, pos_hbm, out_hbm, idx_v, pos_v, tok_v,
                isem, psem, *bufsems):
    wid = lax.axis_index("s") * _NC + lax.axis_index("c")
    p0 = wid * _PPW
    gsem = bufsems[:_NBUF]
    ssem = bufsems[_NBUF:]

    idx_copies = [
        pltpu.async_copy(ids_hbm.at[b, pl.ds(p0, _PPW)], idx_v.at[b], isem)
        for b in range(_B)
    ]
    pos0 = pltpu.async_copy(pos_hbm.at[pl.ds(p0, _CH)], pos_v, psem)
    for h in idx_copies:
        h.wait()

    gh = [None] * _NCHUNK
    sh = [None] * _NCHUNK

    def start_gather(c):
        k, b = divmod(c, _B)
        buf = c % _NBUF
        gh[c] = pltpu.async_copy(
            tok_hbm.at[idx_v.at[b, pl.ds(k * _CH, _CH)]],
            tok_v.at[buf], gsem[buf])

    for c in range(_NBUF - 1):
        start_gather(c)
    pos0.wait()
    for c in range(_NCHUNK):
        buf = c % _NBUF
        k, b = divmod(c, _B)
        gh[c].wait()

        @plsc.parallel_loop(0, _CH, unroll=2)
        def _row(j, buf=buf):
            for cc in range(0, _D, _LANES):
                plsc.addupdate(
                    tok_v.at[buf, j, pl.ds(cc, _LANES)],
                    pos_v[j, pl.ds(cc, _LANES)])

        sh[c] = pltpu.async_copy(
            tok_v.at[buf], out_hbm.at[b, pl.ds(p0 + k * _CH, _CH)], ssem[buf])
        pos1 = None
        if c == _B - 1:
            pos1 = pltpu.async_copy(
                pos_hbm.at[pl.ds(p0 + _CH, _CH)], pos_v, psem)
        nxt = c + _NBUF - 1
        if nxt < _NCHUNK:
            if nxt - _NBUF >= 0:
                sh[nxt - _NBUF].wait()
            start_gather(nxt)
        if pos1 is not None:
            pos1.wait()
    for c in range(_NCHUNK - _NBUF, _NCHUNK):
        sh[c].wait()


def kernel(inputs, W_tok, W_pos):
    return _emb_kernel(W_tok, inputs.astype(jnp.int32), W_pos)


# speedup vs baseline: 1.0556x; 1.0418x over previous
import functools

import jax
import jax.numpy as jnp
from jax import lax
from jax.experimental import pallas as pl
from jax.experimental.pallas import tpu as pltpu
from jax.experimental.pallas import tpu_sc as plsc

_B, _S, _D = 4, 2048, 768
_NC, _NS = 2, 16
_NW = _NC * _NS
_PPW = _S // _NW
_CH = 32
_NCHUNK = _B * _PPW // _CH
_NBUF = 3
_LANES = 16

_mesh = plsc.VectorSubcoreMesh(core_axis_name="c", subcore_axis_name="s")


@functools.partial(
    pl.kernel,
    mesh=_mesh,
    out_type=jax.ShapeDtypeStruct((_B, _S, _D), jnp.float32),
    scratch_types=[
        pltpu.VMEM((_B, _PPW), jnp.int32),
        pltpu.VMEM((_PPW, _D), jnp.float32),
        pltpu.VMEM((_NBUF, _CH, _D), jnp.float32),
        pltpu.SemaphoreType.DMA,
        pltpu.SemaphoreType.DMA,
        pltpu.SemaphoreType.DMA,
        pltpu.SemaphoreType.DMA,
        pltpu.SemaphoreType.DMA,
        pltpu.SemaphoreType.DMA,
        pltpu.SemaphoreType.DMA,
        pltpu.SemaphoreType.DMA,
    ],
)
def _emb_kernel(tok_hbm, ids_hbm, pos_hbm, out_hbm, idx_v, pos_v, tok_v,
                isem, psem, gsem0, gsem1, gsem2, ssem0, ssem1, ssem2):
    wid = lax.axis_index("s") * _NC + lax.axis_index("c")
    p0 = wid * _PPW
    gsem = (gsem0, gsem1, gsem2)
    ssem = (ssem0, ssem1, ssem2)

    idx_copies = [
        pltpu.async_copy(ids_hbm.at[b, pl.ds(p0, _PPW)], idx_v.at[b], isem)
        for b in range(_B)
    ]
    pos_copy = pltpu.async_copy(pos_hbm.at[pl.ds(p0, _PPW)], pos_v, psem)
    for h in idx_copies:
        h.wait()

    gh = [None] * _NCHUNK
    sh = [None] * _NCHUNK

    def start_gather(i):
        b, k = divmod(i, _PPW // _CH)
        buf = i % _NBUF
        gh[i] = pltpu.async_copy(
            tok_hbm.at[idx_v.at[b, pl.ds(k * _CH, _CH)]],
            tok_v.at[buf], gsem[buf])

    start_gather(0)
    start_gather(1)
    pos_copy.wait()
    for i in range(_NCHUNK):
        buf = i % _NBUF
        gh[i].wait()
        b, k = divmod(i, _PPW // _CH)

        @plsc.parallel_loop(0, _CH, unroll=2)
        def _row(j, k=k, buf=buf):
            for cc in range(0, _D, _LANES):
                plsc.addupdate(
                    tok_v.at[buf, j, pl.ds(cc, _LANES)],
                    pos_v[k * _CH + j, pl.ds(cc, _LANES)])

        sh[i] = pltpu.async_copy(
            tok_v.at[buf], out_hbm.at[b, pl.ds(p0 + k * _CH, _CH)], ssem[buf])
        if i + 2 < _NCHUNK:
            if i - 1 >= 0:
                sh[i - 1].wait()
            start_gather(i + 2)
    sh[_NCHUNK - 3].wait()
    sh[_NCHUNK - 2].wait()
    sh[_NCHUNK - 1].wait()


def kernel(inputs, W_tok, W_pos):
    return _emb_kernel(W_tok, inputs.astype(jnp.int32), W_pos)
